# baseline (speedup 1.0000x reference)
import jax
import jax.numpy as jnp
from jax import lax
from jax.experimental import pallas as pl
from jax.experimental.pallas import tpu as pltpu
from jax.experimental.pallas import tpu_sc as plsc

N = 10000
E = 160000
G = 64
DI = 128
DX = 144
DH = 256

NW = 32
EPT = 5120
K = 128
NCH = EPT // K
KA = 80
NCHA = EPT // KA
NPAD = 10240
TRASH = N
CRAWN = G * NPAD
NBLK = 10
BR = NPAD // NBLK


def _sc_a_body(xext, srcl, dstl, acc_out,
               src_v, dst_v, rows0_v, rows1_v, acc_sh,
               gsem0, gsem1, ssem0, ssem1):
    c = lax.axis_index("c")
    s = lax.axis_index("s")
    w = c * 16 + s
    zeros16 = jnp.zeros((16,), jnp.float32)

    pltpu.sync_copy(srcl.at[w], src_v)
    pltpu.sync_copy(dstl.at[w], dst_v)

    def _zrows(t, _):
        rows0_v[t // 9, pl.ds((t % 9) * 16, 16)] = zeros16
        return 0
    lax.fori_loop(0, KA * 9, _zrows, 0)

    def _zacc(i, _):
        pltpu.sync_copy(rows0_v, acc_sh.at[pl.ds(s * 640 + i * KA, KA)])
        return 0
    lax.fori_loop(0, 640 // KA, _zacc, 0)

    pltpu.async_copy(xext.at[src_v.at[0]], rows0_v, gsem0)
    plsc.subcore_barrier()

    @pl.loop(0, NCHA, step=2)
    def _pair(ci):
        pltpu.make_async_copy(xext.at[src_v.at[ci]], rows0_v, gsem0).wait()
        pltpu.async_copy(rows0_v, acc_sh.at[dst_v.at[ci]], ssem0, add=True)

        @pl.when(ci > 0)
        def _():
            pltpu.make_async_copy(rows1_v, acc_sh.at[dst_v.at[ci]],
                                  ssem1).wait()
        pltpu.async_copy(xext.at[src_v.at[ci + 1]], rows1_v, gsem1)
        pltpu.make_async_copy(xext.at[src_v.at[ci]], rows1_v, gsem1).wait()
        pltpu.async_copy(rows1_v, acc_sh.at[dst_v.at[ci + 1]], ssem1, add=True)

        pltpu.make_async_copy(rows0_v, acc_sh.at[dst_v.at[ci]],
                              ssem0).wait()
        nxt = jnp.minimum(ci + 2, NCHA - 1)
        pltpu.async_copy(xext.at[src_v.at[nxt]], rows0_v, gsem0)

    pltpu.make_async_copy(xext.at[src_v.at[0]], rows0_v, gsem0).wait()
    pltpu.make_async_copy(rows1_v, acc_sh.at[dst_v.at[0]],
                          ssem1).wait()
    plsc.subcore_barrier()

    pltpu.sync_copy(acc_sh.at[pl.ds(s * 640, 640)],
                    acc_out.at[c, pl.ds(s * 640, 640)])


def _tc1_body(acc_ref, x_ref, w1l_ref, w1r_ref, b1_ref, h1_ref, invd_ref):
    i = pl.program_id(0)
    agg = acc_ref[0] + acc_ref[1]
    deg = agg[:, DI:DI + 1]
    rows = i * BR + lax.broadcasted_iota(jnp.int32, (BR, 1), 0)
    invd = jnp.where(rows < N, 1.0 / jnp.maximum(deg, 1.0), 0.0)
    mean = agg[:, :DI] * invd
    h1_ref[...] = jnp.maximum(
        jnp.dot(mean, w1l_ref[...], preferred_element_type=jnp.float32)
        + jnp.dot(x_ref[...][:, :DI], w1r_ref[...],
                  preferred_element_type=jnp.float32)
        + b1_ref[...], 0.0)
    invd_ref[...] = invd


def _sc_c_body(srcf, dstf, batchp, invdeg, craw_out,
               src_v, dst_v, rowidx_v, w_v, zed_v, batch_v, invd_v, craw_sh):
    c = lax.axis_index("c")
    s = lax.axis_index("s")
    w = c * 16 + s
    zeros16 = jnp.zeros((16,), jnp.float32)

    pltpu.sync_copy(srcf.at[w], src_v)
    pltpu.sync_copy(dstf.at[w], dst_v)
    pltpu.sync_copy(batchp, batch_v)
    pltpu.sync_copy(invdeg, invd_v)

    def _zed(i, _):
        zed_v[pl.ds(i * 16, 16)] = zeros16
        return 0
    lax.fori_loop(0, 1024 // 16, _zed, 0)

    def _zcraw(i, _):
        pltpu.sync_copy(zed_v, craw_sh.at[pl.ds(s * 40960 + i * 1024, 1024)])
        return 0
    lax.fori_loop(0, 40, _zcraw, 0)
    plsc.subcore_barrier()

    @pl.loop(0, NCH)
    def _chunk(cc):
        for j in range(8):
            d16 = dst_v[pl.ds(cc * K + j * 16, 16)]
            s16 = src_v[pl.ds(cc * K + j * 16, 16)]
            g16 = plsc.load_gather(batch_v, [d16])
            w16 = plsc.load_gather(invd_v, [d16])
            rowidx_v[pl.ds(j * 16, 16)] = jnp.maximum(g16, 0) * NPAD + s16
            w_v[pl.ds(j * 16, 16)] = w16
        pltpu.sync_copy(w_v, craw_sh.at[rowidx_v], add=True)

    plsc.subcore_barrier()

    pltpu.sync_copy(craw_sh.at[pl.ds(s * 40960, 40960)],
                    craw_out.at[c, pl.ds(s * 40960, 40960)])


def _tc2_body(h1_ref, batch_ref, craw_ref, w2l_ref, w2r_ref, b2_ref,
              out_ref, m1_ref, m2_ref, cnt_ref):
    i = pl.program_id(0)

    @pl.when(i == 0)
    def _():
        m1_ref[...] = jnp.zeros_like(m1_ref)
        m2_ref[...] = jnp.zeros_like(m2_ref)
        cnt_ref[...] = jnp.zeros_like(cnt_ref)

    h1 = h1_ref[...]
    craw = craw_ref[0] + craw_ref[1]
    m1_ref[...] += jnp.dot(craw, h1, preferred_element_type=jnp.float32)

    gids = lax.broadcasted_iota(jnp.int32, (G, BR), 0)
    ind = (gids == batch_ref[0]).astype(jnp.float32)
    m2_ref[...] += jnp.dot(ind, h1, preferred_element_type=jnp.float32)
    cnt_ref[:, 0:1] += jnp.sum(ind, axis=1, keepdims=True)

    @pl.when(i == NBLK - 1)
    def _():
        pooled = (jnp.dot(m1_ref[...], w2l_ref[...],
                          preferred_element_type=jnp.float32)
                  + jnp.dot(m2_ref[...], w2r_ref[...],
                            preferred_element_type=jnp.float32))
        out_ref[...] = pooled / jnp.maximum(cnt_ref[:, 0:1], 1.0) + b2_ref[...]


def kernel(x, edge_index, batch, W1_l, W1_r, b1, W2_l, W2_r, b2):
    src = edge_index[0].astype(jnp.int32)
    dst = edge_index[1].astype(jnp.int32)
    srcl = jnp.pad(src.reshape(NW, E // NW),
                   ((0, 0), (0, EPT - E // NW))).reshape(NW, NCH, K)
    dstl = jnp.pad(dst.reshape(NW, E // NW), ((0, 0), (0, EPT - E // NW)),
                   constant_values=TRASH).reshape(NW, NCH, K)
    xext = jnp.pad(jnp.concatenate(
        [x, jnp.ones((N, 1), jnp.float32)], axis=1),
        ((0, NPAD - N), (0, DX - DI - 1)))
    batchp = jnp.pad(batch.astype(jnp.int32), (0, NPAD - N),
                     constant_values=-1)

    mesh = plsc.VectorSubcoreMesh(core_axis_name="c", subcore_axis_name="s")
    sc_params = pltpu.CompilerParams(needs_layout_passes=False,
                                     use_tc_tiling_on_sc=False)

    acc = pl.kernel(
        _sc_a_body,
        compiler_params=sc_params,
        out_type=jax.ShapeDtypeStruct((2, NPAD, DX), jnp.float32),
        mesh=mesh,
        scratch_types=[
            pltpu.VMEM((NCHA, KA), jnp.int32),
            pltpu.VMEM((NCHA, KA), jnp.int32),
            pltpu.VMEM((KA, DX), jnp.float32),
            pltpu.VMEM((KA, DX), jnp.float32),
            pltpu.VMEM_SHARED((NPAD, DX), jnp.float32),
            pltpu.SemaphoreType.DMA,
            pltpu.SemaphoreType.DMA,
            pltpu.SemaphoreType.DMA,
            pltpu.SemaphoreType.DMA,
        ],
    )(xext, srcl.reshape(NW, NCHA, KA), dstl.reshape(NW, NCHA, KA))

    h1, invdeg = pl.pallas_call(
        _tc1_body,
        grid=(NBLK,),
        in_specs=[
            pl.BlockSpec((2, BR, DX), lambda i: (0, i, 0)),
            pl.BlockSpec((BR, DX), lambda i: (i, 0)),
            pl.BlockSpec((DI, DH), lambda i: (0, 0)),
            pl.BlockSpec((DI, DH), lambda i: (0, 0)),
            pl.BlockSpec((1, DH), lambda i: (0, 0)),
        ],
        out_specs=[pl.BlockSpec((BR, DH), lambda i: (i, 0)),
                   pl.BlockSpec((BR, 1), lambda i: (i, 0))],
        out_shape=[jax.ShapeDtypeStruct((NPAD, DH), jnp.float32),
                   jax.ShapeDtypeStruct((NPAD, 1), jnp.float32)],
    )(acc, xext, W1_l, W1_r, b1.reshape(1, DH))

    craw = pl.kernel(
        _sc_c_body,
        compiler_params=sc_params,
        out_type=jax.ShapeDtypeStruct((2, CRAWN), jnp.float32),
        mesh=mesh,
        scratch_types=[
            pltpu.VMEM((EPT,), jnp.int32),
            pltpu.VMEM((EPT,), jnp.int32),
            pltpu.VMEM((K,), jnp.int32),
            pltpu.VMEM((K,), jnp.float32),
            pltpu.VMEM((1024,), jnp.float32),
            pltpu.VMEM((NPAD,), jnp.int32),
            pltpu.VMEM((NPAD,), jnp.float32),
            pltpu.VMEM_SHARED((CRAWN,), jnp.float32),
        ],
    )(srcl.reshape(NW, EPT), dstl.reshape(NW, EPT), batchp,
      invdeg.reshape(NPAD))

    out = pl.pallas_call(
        _tc2_body,
        grid=(NBLK,),
        in_specs=[
            pl.BlockSpec((BR, DH), lambda i: (i, 0)),
            pl.BlockSpec((1, 1, BR), lambda i: (i, 0, 0)),
            pl.BlockSpec((2, G, BR), lambda i: (0, 0, i)),
            pl.BlockSpec((DH, DH), lambda i: (0, 0)),
            pl.BlockSpec((DH, DH), lambda i: (0, 0)),
            pl.BlockSpec((1, DH), lambda i: (0, 0)),
        ],
        out_specs=pl.BlockSpec((G, DH), lambda i: (0, 0)),
        out_shape=jax.ShapeDtypeStruct((G, DH), jnp.float32),
        scratch_shapes=[
            pltpu.VMEM((G, DH), jnp.float32),
            pltpu.VMEM((G, DH), jnp.float32),
            pltpu.VMEM((G, 128), jnp.float32),
        ],
    )(h1, batchp.reshape(NBLK, 1, BR), craw.reshape(2, G, NPAD),
      W2_l, W2_r, b2.reshape(1, DH))
    return out


# speedup vs baseline: 1.2046x; 1.2046x over previous
import jax
import jax.numpy as jnp
from jax import lax
from jax.experimental import pallas as pl
from jax.experimental.pallas import tpu as pltpu
from jax.experimental.pallas import tpu_sc as plsc

N = 10000
E = 160000
G = 64
DI = 128
DX = 144
DH = 256

NW = 32
EPT = 5120
K = 128
NCH = EPT // K
NPAD = 10240
TRASH = N
CRAWN = G * NPAD
NBLK = 10
BR = NPAD // NBLK


def _sc_a_body(xext, srcl, dstl, acc_out,
               src_v, dst_v, rows_v, acc_sh, gsem):
    c = lax.axis_index("c")
    s = lax.axis_index("s")
    w = c * 16 + s
    zeros16 = jnp.zeros((16,), jnp.float32)

    pltpu.sync_copy(srcl.at[w], src_v)
    pltpu.sync_copy(dstl.at[w], dst_v)

    def _zrows(t, _):
        rows_v[t // 9, pl.ds((t % 9) * 16, 16)] = zeros16
        return 0
    lax.fori_loop(0, K * 9, _zrows, 0)

    def _zacc(i, _):
        pltpu.sync_copy(rows_v, acc_sh.at[pl.ds(s * 640 + i * K, K)])
        return 0
    lax.fori_loop(0, 5, _zacc, 0)
    plsc.subcore_barrier()

    @pl.loop(0, NCH)
    def _chunk(ci):
        pltpu.async_copy(xext.at[src_v.at[ci]], rows_v, gsem).wait()
        pltpu.sync_copy(rows_v, acc_sh.at[dst_v.at[ci]], add=True)

    plsc.subcore_barrier()

    pltpu.sync_copy(acc_sh.at[pl.ds(s * 640, 640)],
                    acc_out.at[c, pl.ds(s * 640, 640)])


def _tc1_body(acc_ref, x_ref, w1l_ref, w1r_ref, b1_ref, h1_ref, invd_ref):
    i = pl.program_id(0)
    agg = acc_ref[0] + acc_ref[1]
    deg = agg[:, DI:DI + 1]
    rows = i * BR + lax.broadcasted_iota(jnp.int32, (BR, 1), 0)
    invd = jnp.where(rows < N, 1.0 / jnp.maximum(deg, 1.0), 0.0)
    mean = agg[:, :DI] * invd
    h1_ref[...] = jnp.maximum(
        jnp.dot(mean, w1l_ref[...], preferred_element_type=jnp.float32)
        + jnp.dot(x_ref[...][:, :DI], w1r_ref[...],
                  preferred_element_type=jnp.float32)
        + b1_ref[...], 0.0)
    invd_ref[...] = invd


def _sc_c_body(srcf, dstf, batchp, invdeg, craw_out,
               src_v, dst_v, rowidx_v, w_v, zed_v, batch_v, invd_v, craw_sh):
    c = lax.axis_index("c")
    s = lax.axis_index("s")
    w = c * 16 + s
    zeros16 = jnp.zeros((16,), jnp.float32)

    pltpu.sync_copy(srcf.at[w], src_v)
    pltpu.sync_copy(dstf.at[w], dst_v)
    pltpu.sync_copy(batchp, batch_v)
    pltpu.sync_copy(invdeg, invd_v)

    def _zed(i, _):
        zed_v[pl.ds(i * 16, 16)] = zeros16
        return 0
    lax.fori_loop(0, 1024 // 16, _zed, 0)

    def _zcraw(i, _):
        pltpu.sync_copy(zed_v, craw_sh.at[pl.ds(s * 40960 + i * 1024, 1024)])
        return 0
    lax.fori_loop(0, 40, _zcraw, 0)
    plsc.subcore_barrier()

    @pl.loop(0, NCH)
    def _chunk(cc):
        for j in range(8):
            d16 = dst_v[pl.ds(cc * K + j * 16, 16)]
            s16 = src_v[pl.ds(cc * K + j * 16, 16)]
            g16 = plsc.load_gather(batch_v, [d16])
            w16 = plsc.load_gather(invd_v, [d16])
            rowidx_v[pl.ds(j * 16, 16)] = jnp.maximum(g16, 0) * NPAD + s16
            w_v[pl.ds(j * 16, 16)] = w16
        pltpu.sync_copy(w_v, craw_sh.at[rowidx_v], add=True)

    plsc.subcore_barrier()

    pltpu.sync_copy(craw_sh.at[pl.ds(s * 40960, 40960)],
                    craw_out.at[c, pl.ds(s * 40960, 40960)])


def _tc2_body(h1_ref, batch_ref, craw_ref, w2l_ref, w2r_ref, b2_ref,
              out_ref, m1_ref, m2_ref, cnt_ref):
    i = pl.program_id(0)

    @pl.when(i == 0)
    def _():
        m1_ref[...] = jnp.zeros_like(m1_ref)
        m2_ref[...] = jnp.zeros_like(m2_ref)
        cnt_ref[...] = jnp.zeros_like(cnt_ref)

    h1 = h1_ref[...]
    craw = craw_ref[0] + craw_ref[1]
    m1_ref[...] += jnp.dot(craw, h1, preferred_element_type=jnp.float32)

    gids = lax.broadcasted_iota(jnp.int32, (G, BR), 0)
    ind = (gids == batch_ref[0]).astype(jnp.float32)
    m2_ref[...] += jnp.dot(ind, h1, preferred_element_type=jnp.float32)
    cnt_ref[:, 0:1] += jnp.sum(ind, axis=1, keepdims=True)

    @pl.when(i == NBLK - 1)
    def _():
        pooled = (jnp.dot(m1_ref[...], w2l_ref[...],
                          preferred_element_type=jnp.float32)
                  + jnp.dot(m2_ref[...], w2r_ref[...],
                            preferred_element_type=jnp.float32))
        out_ref[...] = pooled / jnp.maximum(cnt_ref[:, 0:1], 1.0) + b2_ref[...]


def kernel(x, edge_index, batch, W1_l, W1_r, b1, W2_l, W2_r, b2):
    src = edge_index[0].astype(jnp.int32)
    dst = edge_index[1].astype(jnp.int32)
    srcl = jnp.pad(src.reshape(NW, E // NW),
                   ((0, 0), (0, EPT - E // NW))).reshape(NW, NCH, K)
    dstl = jnp.pad(dst.reshape(NW, E // NW), ((0, 0), (0, EPT - E // NW)),
                   constant_values=TRASH).reshape(NW, NCH, K)
    xext = jnp.pad(jnp.concatenate(
        [x, jnp.ones((N, 1), jnp.float32)], axis=1),
        ((0, NPAD - N), (0, DX - DI - 1)))
    batchp = jnp.pad(batch.astype(jnp.int32), (0, NPAD - N),
                     constant_values=-1)

    mesh = plsc.VectorSubcoreMesh(core_axis_name="c", subcore_axis_name="s")
    sc_a_params = pltpu.CompilerParams(needs_layout_passes=False,
                                       use_tc_tiling_on_sc=False)
    sc_c_params = pltpu.CompilerParams(needs_layout_passes=False)

    acc = pl.kernel(
        _sc_a_body,
        compiler_params=sc_a_params,
        out_type=jax.ShapeDtypeStruct((2, NPAD, DX), jnp.float32),
        mesh=mesh,
        scratch_types=[
            pltpu.VMEM((NCH, K), jnp.int32),
            pltpu.VMEM((NCH, K), jnp.int32),
            pltpu.VMEM((K, DX), jnp.float32),
            pltpu.VMEM_SHARED((NPAD, DX), jnp.float32),
            pltpu.SemaphoreType.DMA,
        ],
    )(xext, srcl, dstl)

    h1, invdeg = pl.pallas_call(
        _tc1_body,
        grid=(NBLK,),
        in_specs=[
            pl.BlockSpec((2, BR, DX), lambda i: (0, i, 0)),
            pl.BlockSpec((BR, DX), lambda i: (i, 0)),
            pl.BlockSpec((DI, DH), lambda i: (0, 0)),
            pl.BlockSpec((DI, DH), lambda i: (0, 0)),
            pl.BlockSpec((1, DH), lambda i: (0, 0)),
        ],
        out_specs=[pl.BlockSpec((BR, DH), lambda i: (i, 0)),
                   pl.BlockSpec((BR, 1), lambda i: (i, 0))],
        out_shape=[jax.ShapeDtypeStruct((NPAD, DH), jnp.float32),
                   jax.ShapeDtypeStruct((NPAD, 1), jnp.float32)],
    )(acc, xext, W1_l, W1_r, b1.reshape(1, DH))

    craw = pl.kernel(
        _sc_c_body,
        compiler_params=sc_c_params,
        out_type=jax.ShapeDtypeStruct((2, CRAWN), jnp.float32),
        mesh=mesh,
        scratch_types=[
            pltpu.VMEM((EPT,), jnp.int32),
            pltpu.VMEM((EPT,), jnp.int32),
            pltpu.VMEM((K,), jnp.int32),
            pltpu.VMEM((K,), jnp.float32),
            pltpu.VMEM((1024,), jnp.float32),
            pltpu.VMEM((NPAD,), jnp.int32),
            pltpu.VMEM((NPAD,), jnp.float32),
            pltpu.VMEM_SHARED((CRAWN,), jnp.float32),
        ],
    )(srcl.reshape(NW, EPT), dstl.reshape(NW, EPT), batchp,
      invdeg.reshape(NPAD))

    out = pl.pallas_call(
        _tc2_body,
        grid=(NBLK,),
        in_specs=[
            pl.BlockSpec((BR, DH), lambda i: (i, 0)),
            pl.BlockSpec((1, 1, BR), lambda i: (i, 0, 0)),
            pl.BlockSpec((2, G, BR), lambda i: (0, 0, i)),
            pl.BlockSpec((DH, DH), lambda i: (0, 0)),
            pl.BlockSpec((DH, DH), lambda i: (0, 0)),
            pl.BlockSpec((1, DH), lambda i: (0, 0)),
        ],
        out_specs=pl.BlockSpec((G, DH), lambda i: (0, 0)),
        out_shape=jax.ShapeDtypeStruct((G, DH), jnp.float32),
        scratch_shapes=[
            pltpu.VMEM((G, DH), jnp.float32),
            pltpu.VMEM((G, DH), jnp.float32),
            pltpu.VMEM((G, 128), jnp.float32),
        ],
    )(h1, batchp.reshape(NBLK, 1, BR), craw.reshape(2, G, NPAD),
      W2_l, W2_r, b2.reshape(1, DH))
    return out


# speedup vs baseline: 1.2410x; 1.0303x over previous
import jax
import jax.numpy as jnp
from jax import lax
from jax.experimental import pallas as pl
from jax.experimental.pallas import tpu as pltpu
from jax.experimental.pallas import tpu_sc as plsc

N = 10000
E = 160000
G = 64
DI = 128
DX = 144
DH = 256

NW = 32
EPT = 5120
K = 128
NCH = EPT // K
NPAD = 10240
TRASH = N
CRAWN = G * NPAD
NBLK = 10
BR = NPAD // NBLK


def _sc_a_body(xext, srcl, dstl, acc_out, deg_out,
               src_v, dst_v, rows_v, acc_sh, gsem):
    c = lax.axis_index("c")
    s = lax.axis_index("s")
    w = c * 16 + s
    zeros16 = jnp.zeros((16,), jnp.float32)

    pltpu.sync_copy(srcl.at[w], src_v)
    pltpu.sync_copy(dstl.at[w], dst_v)

    def _zrows(t, _):
        rows_v[t // 9, pl.ds((t % 9) * 16, 16)] = zeros16
        return 0
    lax.fori_loop(0, K * 9, _zrows, 0)

    def _zacc(i, _):
        pltpu.sync_copy(rows_v, acc_sh.at[pl.ds(s * 640 + i * K, K)])
        return 0
    lax.fori_loop(0, 5, _zacc, 0)
    plsc.subcore_barrier()

    @pl.loop(0, NCH)
    def _chunk(ci):
        pltpu.async_copy(xext.at[src_v.at[ci]], rows_v, gsem).wait()
        pltpu.sync_copy(rows_v, acc_sh.at[dst_v.at[ci]], add=True)

    plsc.subcore_barrier()

    pltpu.sync_copy(acc_sh.at[pl.ds(s * 640, 640), pl.ds(0, DI)],
                    acc_out.at[c, pl.ds(s * 640, 640)])
    pltpu.sync_copy(acc_sh.at[pl.ds(s * 640, 640), pl.ds(DI, 16)],
                    deg_out.at[c, pl.ds(s * 640, 640)])


def _tc1_body(acc_ref, deg_ref, x_ref, w1l_ref, w1r_ref, b1_ref,
              h1_ref, invd_ref):
    i = pl.program_id(0)
    agg = acc_ref[0] + acc_ref[1]
    deg = deg_ref[0][:, 0:1] + deg_ref[1][:, 0:1]
    rows = i * BR + lax.broadcasted_iota(jnp.int32, (BR, 1), 0)
    invd = jnp.where(rows < N, 1.0 / jnp.maximum(deg, 1.0), 0.0)
    mean = agg * invd
    h1_ref[...] = jnp.maximum(
        jnp.dot(mean, w1l_ref[...], preferred_element_type=jnp.float32)
        + jnp.dot(x_ref[...][:, :DI], w1r_ref[...],
                  preferred_element_type=jnp.float32)
        + b1_ref[...], 0.0)
    invd_ref[...] = invd


def _sc_c_body(srcf, dstf, batchp, invdeg, craw_out,
               src_v, dst_v, rowidx_v, w_v, zed_v, batch_v, invd_v, craw_sh):
    c = lax.axis_index("c")
    s = lax.axis_index("s")
    w = c * 16 + s
    zeros16 = jnp.zeros((16,), jnp.float32)

    pltpu.sync_copy(srcf.at[w], src_v)
    pltpu.sync_copy(dstf.at[w], dst_v)
    pltpu.sync_copy(batchp, batch_v)
    pltpu.sync_copy(invdeg, invd_v)

    def _zed(i, _):
        zed_v[pl.ds(i * 16, 16)] = zeros16
        return 0
    lax.fori_loop(0, 1024 // 16, _zed, 0)

    def _zcraw(i, _):
        pltpu.sync_copy(zed_v, craw_sh.at[pl.ds(s * 40960 + i * 1024, 1024)])
        return 0
    lax.fori_loop(0, 40, _zcraw, 0)
    plsc.subcore_barrier()

    @pl.loop(0, NCH)
    def _chunk(cc):
        for j in range(8):
            d16 = dst_v[pl.ds(cc * K + j * 16, 16)]
            s16 = src_v[pl.ds(cc * K + j * 16, 16)]
            g16 = plsc.load_gather(batch_v, [d16])
            w16 = plsc.load_gather(invd_v, [d16])
            rowidx_v[pl.ds(j * 16, 16)] = jnp.maximum(g16, 0) * NPAD + s16
            w_v[pl.ds(j * 16, 16)] = w16
        pltpu.sync_copy(w_v, craw_sh.at[rowidx_v], add=True)

    plsc.subcore_barrier()

    pltpu.sync_copy(craw_sh.at[pl.ds(s * 40960, 40960)],
                    craw_out.at[c, pl.ds(s * 40960, 40960)])


def _tc2_body(h1_ref, batch_ref, craw_ref, w2l_ref, w2r_ref, b2_ref,
              out_ref, m1_ref, m2_ref, cnt_ref):
    i = pl.program_id(0)

    @pl.when(i == 0)
    def _():
        m1_ref[...] = jnp.zeros_like(m1_ref)
        m2_ref[...] = jnp.zeros_like(m2_ref)
        cnt_ref[...] = jnp.zeros_like(cnt_ref)

    h1 = h1_ref[...]
    craw = craw_ref[0] + craw_ref[1]
    m1_ref[...] += jnp.dot(craw, h1, preferred_element_type=jnp.float32)

    gids = lax.broadcasted_iota(jnp.int32, (G, BR), 0)
    ind = (gids == batch_ref[0]).astype(jnp.float32)
    m2_ref[...] += jnp.dot(ind, h1, preferred_element_type=jnp.float32)
    cnt_ref[:, 0:1] += jnp.sum(ind, axis=1, keepdims=True)

    @pl.when(i == NBLK - 1)
    def _():
        pooled = (jnp.dot(m1_ref[...], w2l_ref[...],
                          preferred_element_type=jnp.float32)
                  + jnp.dot(m2_ref[...], w2r_ref[...],
                            preferred_element_type=jnp.float32))
        out_ref[...] = pooled / jnp.maximum(cnt_ref[:, 0:1], 1.0) + b2_ref[...]


def kernel(x, edge_index, batch, W1_l, W1_r, b1, W2_l, W2_r, b2):
    src = edge_index[0].astype(jnp.int32)
    dst = edge_index[1].astype(jnp.int32)
    srcl = jnp.pad(src.reshape(NW, E // NW),
                   ((0, 0), (0, EPT - E // NW))).reshape(NW, NCH, K)
    dstl = jnp.pad(dst.reshape(NW, E // NW), ((0, 0), (0, EPT - E // NW)),
                   constant_values=TRASH).reshape(NW, NCH, K)
    xext = jnp.pad(jnp.concatenate(
        [x, jnp.ones((N, 1), jnp.float32)], axis=1),
        ((0, NPAD - N), (0, DX - DI - 1)))
    batchp = jnp.pad(batch.astype(jnp.int32), (0, NPAD - N),
                     constant_values=-1)

    mesh = plsc.VectorSubcoreMesh(core_axis_name="c", subcore_axis_name="s")
    sc_a_params = pltpu.CompilerParams(needs_layout_passes=False,
                                       use_tc_tiling_on_sc=False)
    sc_c_params = pltpu.CompilerParams(needs_layout_passes=False)

    acc, accdeg = pl.kernel(
        _sc_a_body,
        compiler_params=sc_a_params,
        out_type=(jax.ShapeDtypeStruct((2, NPAD, DI), jnp.float32),
                  jax.ShapeDtypeStruct((2, NPAD, 16), jnp.float32)),
        mesh=mesh,
        scratch_types=[
            pltpu.VMEM((NCH, K), jnp.int32),
            pltpu.VMEM((NCH, K), jnp.int32),
            pltpu.VMEM((K, DX), jnp.float32),
            pltpu.VMEM_SHARED((NPAD, DX), jnp.float32),
            pltpu.SemaphoreType.DMA,
        ],
    )(xext, srcl, dstl)

    h1, invdeg = pl.pallas_call(
        _tc1_body,
        grid=(NBLK,),
        in_specs=[
            pl.BlockSpec((2, BR, DI), lambda i: (0, i, 0)),
            pl.BlockSpec((2, BR, 16), lambda i: (0, i, 0)),
            pl.BlockSpec((BR, DX), lambda i: (i, 0)),
            pl.BlockSpec((DI, DH), lambda i: (0, 0)),
            pl.BlockSpec((DI, DH), lambda i: (0, 0)),
            pl.BlockSpec((1, DH), lambda i: (0, 0)),
        ],
        out_specs=[pl.BlockSpec((BR, DH), lambda i: (i, 0)),
                   pl.BlockSpec((BR, 1), lambda i: (i, 0))],
        out_shape=[jax.ShapeDtypeStruct((NPAD, DH), jnp.float32),
                   jax.ShapeDtypeStruct((NPAD, 1), jnp.float32)],
    )(acc, accdeg, xext, W1_l, W1_r, b1.reshape(1, DH))

    craw = pl.kernel(
        _sc_c_body,
        compiler_params=sc_c_params,
        out_type=jax.ShapeDtypeStruct((2, CRAWN), jnp.float32),
        mesh=mesh,
        scratch_types=[
            pltpu.VMEM((EPT,), jnp.int32),
            pltpu.VMEM((EPT,), jnp.int32),
            pltpu.VMEM((K,), jnp.int32),
            pltpu.VMEM((K,), jnp.float32),
            pltpu.VMEM((1024,), jnp.float32),
            pltpu.VMEM((NPAD,), jnp.int32),
            pltpu.VMEM((NPAD,), jnp.float32),
            pltpu.VMEM_SHARED((CRAWN,), jnp.float32),
        ],
    )(srcl.reshape(NW, EPT), dstl.reshape(NW, EPT), batchp,
      invdeg.reshape(NPAD))

    out = pl.pallas_call(
        _tc2_body,
        grid=(NBLK,),
        in_specs=[
            pl.BlockSpec((BR, DH), lambda i: (i, 0)),
            pl.BlockSpec((1, 1, BR), lambda i: (i, 0, 0)),
            pl.BlockSpec((2, G, BR), lambda i: (0, 0, i)),
            pl.BlockSpec((DH, DH), lambda i: (0, 0)),
            pl.BlockSpec((DH, DH), lambda i: (0, 0)),
            pl.BlockSpec((1, DH), lambda i: (0, 0)),
        ],
        out_specs=pl.BlockSpec((G, DH), lambda i: (0, 0)),
        out_shape=jax.ShapeDtypeStruct((G, DH), jnp.float32),
        scratch_shapes=[
            pltpu.VMEM((G, DH), jnp.float32),
            pltpu.VMEM((G, DH), jnp.float32),
            pltpu.VMEM((G, 128), jnp.float32),
        ],
    )(h1, batchp.reshape(NBLK, 1, BR), craw.reshape(2, G, NPAD),
      W2_l, W2_r, b2.reshape(1, DH))
    return out


# speedup vs baseline: 2.1218x; 1.7097x over previous
import jax
import jax.numpy as jnp
from jax import lax
from jax.experimental import pallas as pl
from jax.experimental.pallas import tpu as pltpu
from jax.experimental.pallas import tpu_sc as plsc

N = 10000
E = 160000
G = 64
DI = 128
DX = 144
DH = 256

NW = 32
EPT = 5000
K = 125
NCH = EPT // K
NPAD = 10240
TRASH = N
CRAWN = G * NPAD
NBLK = 10
BR = NPAD // NBLK


def _sc_a_body(xext, srcl, dstl, acc_out, deg_out,
               src_v, dst_v, rows_v, acc_sh, gsem):
    c = lax.axis_index("c")
    s = lax.axis_index("s")
    w = c * 16 + s
    zeros16 = jnp.zeros((16,), jnp.float32)

    pltpu.sync_copy(srcl.at[w], src_v)
    pltpu.sync_copy(dstl.at[w], dst_v)

    def _zrows(t, _):
        rows_v[t // 9, pl.ds((t % 9) * 16, 16)] = zeros16
        return 0
    lax.fori_loop(0, K * 9, _zrows, 0)

    def _zacc(i, _):
        pltpu.sync_copy(rows_v, acc_sh.at[pl.ds(s * 640 + i * K, K)])
        return 0
    lax.fori_loop(0, 5, _zacc, 0)
    pltpu.sync_copy(rows_v.at[pl.ds(0, 15)],
                    acc_sh.at[pl.ds(s * 640 + 625, 15)])
    plsc.subcore_barrier()

    @pl.loop(0, NCH)
    def _chunk(ci):
        pltpu.async_copy(xext.at[src_v.at[ci]], rows_v, gsem).wait()
        pltpu.sync_copy(rows_v, acc_sh.at[dst_v.at[ci]], add=True)

    plsc.subcore_barrier()

    pltpu.sync_copy(acc_sh.at[pl.ds(s * 640, 640), pl.ds(0, DI)],
                    acc_out.at[c, pl.ds(s * 640, 640)])
    pltpu.sync_copy(acc_sh.at[pl.ds(s * 640, 640), pl.ds(DI, 16)],
                    deg_out.at[c, pl.ds(s * 640, 640)])


def _tc1_body(acc_ref, deg_ref, x_ref, w1l_ref, w1r_ref, b1_ref,
              h1_ref, invd_ref):
    i = pl.program_id(0)
    agg = acc_ref[0] + acc_ref[1]
    deg = deg_ref[0][:, 0:1] + deg_ref[1][:, 0:1]
    rows = i * BR + lax.broadcasted_iota(jnp.int32, (BR, 1), 0)
    invd = jnp.where(rows < N, 1.0 / jnp.maximum(deg, 1.0), 0.0)
    mean = agg * invd
    h1_ref[...] = jnp.maximum(
        jnp.dot(mean, w1l_ref[...], preferred_element_type=jnp.float32)
        + jnp.dot(x_ref[...][:, :DI], w1r_ref[...],
                  preferred_element_type=jnp.float32)
        + b1_ref[...], 0.0)
    invd_ref[...] = invd


def _sc_c_body(srcf, dstf, batchp, invdeg, craw_out,
               src_v, dst_v, rowidx_v, w_v, zed_v, batch_v, invd_v, craw_sh):
    c = lax.axis_index("c")
    s = lax.axis_index("s")
    w = c * 16 + s
    zeros16 = jnp.zeros((16,), jnp.float32)

    pltpu.sync_copy(srcf.at[pl.ds(w * EPT, EPT)], src_v.at[pl.ds(0, EPT)])
    pltpu.sync_copy(dstf.at[pl.ds(w * EPT, EPT)], dst_v.at[pl.ds(0, EPT)])
    pltpu.sync_copy(batchp, batch_v)
    pltpu.sync_copy(invdeg, invd_v)

    def _zed(i, _):
        zed_v[pl.ds(i * 16, 16)] = zeros16
        return 0
    lax.fori_loop(0, 1024 // 16, _zed, 0)

    def _zcraw(i, _):
        pltpu.sync_copy(zed_v, craw_sh.at[pl.ds(s * 40960 + i * 1024, 1024)])
        return 0
    lax.fori_loop(0, 40, _zcraw, 0)
    plsc.subcore_barrier()

    lane16 = lax.iota(jnp.int32, 16)
    zeros16f = jnp.zeros((16,), jnp.float32)
    zeros16i = jnp.zeros((16,), jnp.int32)

    @pl.loop(0, NCH)
    def _chunk(cc):
        for j in range(8):
            d16 = dst_v[pl.ds(cc * K + j * 16, 16)]
            s16 = src_v[pl.ds(cc * K + j * 16, 16)]
            if j == 7:
                tm = lane16 < (K - 112)
                d16 = jnp.where(tm, d16, zeros16i)
                s16 = jnp.where(tm, s16, zeros16i)
            g16 = plsc.load_gather(batch_v, [d16])
            w16 = plsc.load_gather(invd_v, [d16])
            if j == 7:
                w16 = jnp.where(lane16 < (K - 112), w16, zeros16f)
            rowidx_v[pl.ds(j * 16, 16)] = jnp.maximum(g16, 0) * NPAD + s16
            w_v[pl.ds(j * 16, 16)] = w16
        pltpu.sync_copy(w_v, craw_sh.at[rowidx_v], add=True)

    plsc.subcore_barrier()

    for k in range(4):
        pltpu.sync_copy(craw_sh.at[pl.ds((s * 4 + k) * NPAD, NPAD)],
                        craw_out.at[c, s * 4 + k])


def _tc2_body(h1_ref, batch_ref, craw_ref, w2l_ref, w2r_ref, b2_ref,
              out_ref, m1_ref, m2_ref, cnt_ref):
    i = pl.program_id(0)

    @pl.when(i == 0)
    def _():
        m1_ref[...] = jnp.zeros_like(m1_ref)
        m2_ref[...] = jnp.zeros_like(m2_ref)
        cnt_ref[...] = jnp.zeros_like(cnt_ref)

    h1 = h1_ref[...]
    craw = craw_ref[0] + craw_ref[1]
    m1_ref[...] += jnp.dot(craw, h1, preferred_element_type=jnp.float32)

    gids = lax.broadcasted_iota(jnp.int32, (G, BR), 0)
    ind = (gids == batch_ref[0]).astype(jnp.float32)
    m2_ref[...] += jnp.dot(ind, h1, preferred_element_type=jnp.float32)
    cnt_ref[:, 0:1] += jnp.sum(ind, axis=1, keepdims=True)

    @pl.when(i == NBLK - 1)
    def _():
        pooled = (jnp.dot(m1_ref[...], w2l_ref[...],
                          preferred_element_type=jnp.float32)
                  + jnp.dot(m2_ref[...], w2r_ref[...],
                            preferred_element_type=jnp.float32))
        out_ref[...] = pooled / jnp.maximum(cnt_ref[:, 0:1], 1.0) + b2_ref[...]


def kernel(x, edge_index, batch, W1_l, W1_r, b1, W2_l, W2_r, b2):
    src = edge_index[0].astype(jnp.int32)
    dst = edge_index[1].astype(jnp.int32)
    srcl = src.reshape(NW, NCH, K)
    dstl = dst.reshape(NW, NCH, K)
    xext = jnp.pad(jnp.concatenate(
        [x, jnp.ones((N, 1), jnp.float32)], axis=1),
        ((0, NPAD - N), (0, DX - DI - 1)))
    batchp = jnp.pad(batch.astype(jnp.int32), (0, NPAD - N),
                     constant_values=-1)

    mesh = plsc.VectorSubcoreMesh(core_axis_name="c", subcore_axis_name="s")
    sc_a_params = pltpu.CompilerParams(needs_layout_passes=False,
                                       use_tc_tiling_on_sc=False)
    sc_c_params = pltpu.CompilerParams(needs_layout_passes=False)

    acc, accdeg = pl.kernel(
        _sc_a_body,
        compiler_params=sc_a_params,
        out_type=(jax.ShapeDtypeStruct((2, NPAD, DI), jnp.float32),
                  jax.ShapeDtypeStruct((2, NPAD, 16), jnp.float32)),
        mesh=mesh,
        scratch_types=[
            pltpu.VMEM((NCH, K), jnp.int32),
            pltpu.VMEM((NCH, K), jnp.int32),
            pltpu.VMEM((K, DX), jnp.float32),
            pltpu.VMEM_SHARED((NPAD, DX), jnp.float32),
            pltpu.SemaphoreType.DMA,
        ],
    )(xext, srcl, dstl)

    h1, invdeg = pl.pallas_call(
        _tc1_body,
        grid=(NBLK,),
        in_specs=[
            pl.BlockSpec((2, BR, DI), lambda i: (0, i, 0)),
            pl.BlockSpec((2, BR, 16), lambda i: (0, i, 0)),
            pl.BlockSpec((BR, DX), lambda i: (i, 0)),
            pl.BlockSpec((DI, DH), lambda i: (0, 0)),
            pl.BlockSpec((DI, DH), lambda i: (0, 0)),
            pl.BlockSpec((1, DH), lambda i: (0, 0)),
        ],
        out_specs=[pl.BlockSpec((BR, DH), lambda i: (i, 0)),
                   pl.BlockSpec((BR, 1), lambda i: (i, 0))],
        out_shape=[jax.ShapeDtypeStruct((NPAD, DH), jnp.float32),
                   jax.ShapeDtypeStruct((NPAD, 1), jnp.float32)],
    )(acc, accdeg, xext, W1_l, W1_r, b1.reshape(1, DH))

    craw = pl.kernel(
        _sc_c_body,
        compiler_params=sc_c_params,
        out_type=jax.ShapeDtypeStruct((2, G, NPAD), jnp.float32),
        mesh=mesh,
        scratch_types=[
            pltpu.VMEM((EPT + 8,), jnp.int32),
            pltpu.VMEM((EPT + 8,), jnp.int32),
            pltpu.VMEM((128,), jnp.int32),
            pltpu.VMEM((128,), jnp.float32),
            pltpu.VMEM((1024,), jnp.float32),
            pltpu.VMEM((NPAD,), jnp.int32),
            pltpu.VMEM((NPAD,), jnp.float32),
            pltpu.VMEM_SHARED((CRAWN,), jnp.float32),
        ],
    )(src, dst, batchp, invdeg.reshape(NPAD))

    out = pl.pallas_call(
        _tc2_body,
        grid=(NBLK,),
        in_specs=[
            pl.BlockSpec((BR, DH), lambda i: (i, 0)),
            pl.BlockSpec((1, 1, BR), lambda i: (i, 0, 0)),
            pl.BlockSpec((2, G, BR), lambda i: (0, 0, i)),
            pl.BlockSpec((DH, DH), lambda i: (0, 0)),
            pl.BlockSpec((DH, DH), lambda i: (0, 0)),
            pl.BlockSpec((1, DH), lambda i: (0, 0)),
        ],
        out_specs=pl.BlockSpec((G, DH), lambda i: (0, 0)),
        out_shape=jax.ShapeDtypeStruct((G, DH), jnp.float32),
        scratch_shapes=[
            pltpu.VMEM((G, DH), jnp.float32),
            pltpu.VMEM((G, DH), jnp.float32),
            pltpu.VMEM((G, 128), jnp.float32),
        ],
    )(h1, batchp.reshape(NBLK, 1, BR), craw,
      W2_l, W2_r, b2.reshape(1, DH))
    return out


# speedup vs baseline: 2.1279x; 1.0029x over previous
import jax
import jax.numpy as jnp
from jax import lax
from jax.experimental import pallas as pl
from jax.experimental.pallas import tpu as pltpu
from jax.experimental.pallas import tpu_sc as plsc

N = 10000
E = 160000
G = 64
DI = 128
DX = 144
DH = 256

NW = 32
EPT = 5000
K = 125
NCH = EPT // K
NPAD = 10240
CRAWN = G * NPAD
NBLK = 10
BR = NPAD // NBLK


def _sc_a_body(xext, srcl, dstl, acc_out, deg_out,
               src_v, dst_v, rows_v, acc_sh, gsem):
    c = lax.axis_index("c")
    s = lax.axis_index("s")
    w = c * 16 + s
    zeros16 = jnp.zeros((16,), jnp.float32)

    pltpu.sync_copy(srcl.at[w], src_v)
    pltpu.sync_copy(dstl.at[w], dst_v)

    def _zrows(t, _):
        rows_v[t // 9, pl.ds((t % 9) * 16, 16)] = zeros16
        return 0
    lax.fori_loop(0, K * 9, _zrows, 0)

    def _zacc(i, _):
        pltpu.sync_copy(rows_v, acc_sh.at[pl.ds(s * 640 + i * K, K)])
        return 0
    lax.fori_loop(0, 5, _zacc, 0)
    pltpu.sync_copy(rows_v.at[pl.ds(0, 15)],
                    acc_sh.at[pl.ds(s * 640 + 625, 15)])
    plsc.subcore_barrier()

    @pl.loop(0, NCH)
    def _chunk(ci):
        pltpu.async_copy(xext.at[src_v.at[ci]], rows_v, gsem).wait()
        pltpu.sync_copy(rows_v, acc_sh.at[dst_v.at[ci]], add=True)

    plsc.subcore_barrier()

    pltpu.sync_copy(acc_sh.at[pl.ds(s * 640, 640), pl.ds(0, DI)],
                    acc_out.at[c, pl.ds(s * 640, 640)])
    pltpu.sync_copy(acc_sh.at[pl.ds(s * 640, 640), pl.ds(DI, 16)],
                    deg_out.at[c, pl.ds(s * 640, 640)])


def _tc1_body(acc_ref, deg_ref, x_ref, w1l_ref, w1r_ref, b1_ref,
              h1_ref, invd_ref):
    i = pl.program_id(0)
    agg = acc_ref[0] + acc_ref[1]
    deg = deg_ref[0][:, 0:1] + deg_ref[1][:, 0:1]
    rows = i * BR + lax.broadcasted_iota(jnp.int32, (BR, 1), 0)
    invd = jnp.where(rows < N, 1.0 / jnp.maximum(deg, 1.0), 0.0)
    mean = agg * invd
    h1_ref[...] = jnp.maximum(
        jnp.dot(mean, w1l_ref[...], preferred_element_type=jnp.float32)
        + jnp.dot(x_ref[...][:, :DI], w1r_ref[...],
                  preferred_element_type=jnp.float32)
        + b1_ref[...], 0.0)
    invd_ref[...] = invd


def _sc_c_body(srcf, dstf, batchp, invdeg, craw_out,
               src_v, dst_v, rowidx_v, w_v, zed_v, batch_v, invd_v, craw_sh):
    c = lax.axis_index("c")
    s = lax.axis_index("s")
    w = c * 16 + s
    zeros16 = jnp.zeros((16,), jnp.float32)

    pltpu.sync_copy(srcf.at[pl.ds(w * EPT, EPT)], src_v.at[pl.ds(0, EPT)])
    pltpu.sync_copy(dstf.at[pl.ds(w * EPT, EPT)], dst_v.at[pl.ds(0, EPT)])
    pltpu.sync_copy(batchp, batch_v)
    pltpu.sync_copy(invdeg, invd_v)

    def _zed(i, _):
        zed_v[pl.ds(i * 16, 16)] = zeros16
        return 0
    lax.fori_loop(0, 1024 // 16, _zed, 0)

    def _zcraw(i, _):
        pltpu.sync_copy(zed_v, craw_sh.at[pl.ds(s * 40960 + i * 1024, 1024)])
        return 0
    lax.fori_loop(0, 40, _zcraw, 0)
    plsc.subcore_barrier()

    lane16 = lax.iota(jnp.int32, 16)
    zeros16f = jnp.zeros((16,), jnp.float32)
    zeros16i = jnp.zeros((16,), jnp.int32)

    @pl.loop(0, NCH)
    def _chunk(cc):
        for j in range(8):
            d16 = dst_v[pl.ds(cc * K + j * 16, 16)]
            s16 = src_v[pl.ds(cc * K + j * 16, 16)]
            if j == 7:
                tm = lane16 < (K - 112)
                d16 = jnp.where(tm, d16, zeros16i)
                s16 = jnp.where(tm, s16, zeros16i)
            g16 = plsc.load_gather(batch_v, [d16])
            w16 = plsc.load_gather(invd_v, [d16])
            if j == 7:
                w16 = jnp.where(lane16 < (K - 112), w16, zeros16f)
            rowidx_v[pl.ds(j * 16, 16)] = jnp.maximum(g16, 0) * NPAD + s16
            w_v[pl.ds(j * 16, 16)] = w16
        pltpu.sync_copy(w_v, craw_sh.at[rowidx_v], add=True)

    plsc.subcore_barrier()

    for k in range(4):
        pltpu.sync_copy(craw_sh.at[pl.ds((s * 4 + k) * NPAD, NPAD)],
                        craw_out.at[c, s * 4 + k])


def _tc2_body(h1_ref, batch_ref, craw_ref, w2l_ref, w2r_ref, b2_ref,
              out_ref, m1_ref, m2_ref, cnt_ref):
    i = pl.program_id(0)

    @pl.when(i == 0)
    def _():
        m1_ref[...] = jnp.zeros_like(m1_ref)
        m2_ref[...] = jnp.zeros_like(m2_ref)
        cnt_ref[...] = jnp.zeros_like(cnt_ref)

    h1 = h1_ref[...]
    craw = craw_ref[0] + craw_ref[1]
    m1_ref[...] += jnp.dot(craw, h1, preferred_element_type=jnp.float32)

    gids = lax.broadcasted_iota(jnp.int32, (G, BR), 0)
    ind = (gids == batch_ref[0]).astype(jnp.float32)
    m2_ref[...] += jnp.dot(ind, h1, preferred_element_type=jnp.float32)
    cnt_ref[:, 0:1] += jnp.sum(ind, axis=1, keepdims=True)

    @pl.when(i == NBLK - 1)
    def _():
        pooled = (jnp.dot(m1_ref[...], w2l_ref[...],
                          preferred_element_type=jnp.float32)
                  + jnp.dot(m2_ref[...], w2r_ref[...],
                            preferred_element_type=jnp.float32))
        out_ref[...] = pooled / jnp.maximum(cnt_ref[:, 0:1], 1.0) + b2_ref[...]


def kernel(x, edge_index, batch, W1_l, W1_r, b1, W2_l, W2_r, b2):
    src = edge_index[0].astype(jnp.int32)
    dst = edge_index[1].astype(jnp.int32)
    srcl = src.reshape(NW, NCH, K)
    dstl = dst.reshape(NW, NCH, K)
    xext = jnp.pad(jnp.concatenate(
        [x, jnp.ones((N, 1), jnp.float32)], axis=1),
        ((0, NPAD - N), (0, DX - DI - 1)))
    batchp = jnp.pad(batch.astype(jnp.int32), (0, NPAD - N),
                     constant_values=-1)

    mesh = plsc.VectorSubcoreMesh(core_axis_name="c", subcore_axis_name="s")
    sc_a_params = pltpu.CompilerParams(needs_layout_passes=False,
                                       use_tc_tiling_on_sc=False)
    sc_c_params = pltpu.CompilerParams(needs_layout_passes=False)

    acc, accdeg = pl.kernel(
        _sc_a_body,
        compiler_params=sc_a_params,
        out_type=(jax.ShapeDtypeStruct((2, NPAD, DI), jnp.float32),
                  jax.ShapeDtypeStruct((2, NPAD, 16), jnp.float32)),
        mesh=mesh,
        scratch_types=[
            pltpu.VMEM((NCH, K), jnp.int32),
            pltpu.VMEM((NCH, K), jnp.int32),
            pltpu.VMEM((K, DX), jnp.float32),
            pltpu.VMEM_SHARED((NPAD, DX), jnp.float32),
            pltpu.SemaphoreType.DMA,
        ],
    )(xext, srcl, dstl)

    h1, invdeg = pl.pallas_call(
        _tc1_body,
        grid=(NBLK,),
        in_specs=[
            pl.BlockSpec((2, BR, DI), lambda i: (0, i, 0)),
            pl.BlockSpec((2, BR, 16), lambda i: (0, i, 0)),
            pl.BlockSpec((BR, DX), lambda i: (i, 0)),
            pl.BlockSpec((DI, DH), lambda i: (0, 0)),
            pl.BlockSpec((DI, DH), lambda i: (0, 0)),
            pl.BlockSpec((1, DH), lambda i: (0, 0)),
        ],
        out_specs=[pl.BlockSpec((BR, DH), lambda i: (i, 0)),
                   pl.BlockSpec((BR, 1), lambda i: (i, 0))],
        out_shape=[jax.ShapeDtypeStruct((NPAD, DH), jnp.float32),
                   jax.ShapeDtypeStruct((NPAD, 1), jnp.float32)],
    )(acc, accdeg, xext, W1_l, W1_r, b1.reshape(1, DH))

    craw = pl.kernel(
        _sc_c_body,
        compiler_params=sc_c_params,
        out_type=jax.ShapeDtypeStruct((2, G, NPAD), jnp.float32),
        mesh=mesh,
        scratch_types=[
            pltpu.VMEM((EPT + 8,), jnp.int32),
            pltpu.VMEM((EPT + 8,), jnp.int32),
            pltpu.VMEM((128,), jnp.int32),
            pltpu.VMEM((128,), jnp.float32),
            pltpu.VMEM((1024,), jnp.float32),
            pltpu.VMEM((NPAD,), jnp.int32),
            pltpu.VMEM((NPAD,), jnp.float32),
            pltpu.VMEM_SHARED((CRAWN,), jnp.float32),
        ],
    )(src, dst, batchp, invdeg.reshape(NPAD))

    out = pl.pallas_call(
        _tc2_body,
        grid=(NBLK,),
        in_specs=[
            pl.BlockSpec((BR, DH), lambda i: (i, 0)),
            pl.BlockSpec((1, 1, BR), lambda i: (i, 0, 0)),
            pl.BlockSpec((2, G, BR), lambda i: (0, 0, i)),
            pl.BlockSpec((DH, DH), lambda i: (0, 0)),
            pl.BlockSpec((DH, DH), lambda i: (0, 0)),
            pl.BlockSpec((1, DH), lambda i: (0, 0)),
        ],
        out_specs=pl.BlockSpec((G, DH), lambda i: (0, 0)),
        out_shape=jax.ShapeDtypeStruct((G, DH), jnp.float32),
        scratch_shapes=[
            pltpu.VMEM((G, DH), jnp.float32),
            pltpu.VMEM((G, DH), jnp.float32),
            pltpu.VMEM((G, 128), jnp.float32),
        ],
    )(h1, batchp.reshape(NBLK, 1, BR), craw,
      W2_l, W2_r, b2.reshape(1, DH))
    return out
